# baseline (speedup 1.0000x reference)
import jax
import jax.numpy as jnp
from jax import lax
from jax.experimental import pallas as pl
from jax.experimental.pallas import tpu as pltpu
from jax.experimental.pallas import tpu_sc as plsc

N_ITEMS = 100000
EMBED_DIM = 16
PACKED_DIM = EMBED_DIM // 2
N_EDGES = 3200000

NUM_CORES = 2
NUM_SUBCORES = 16
NUM_WORKERS = NUM_CORES * NUM_SUBCORES

CHUNK = 2000
N_CHUNKS = N_EDGES // CHUNK
GROUPS = CHUNK // 16
GROUP_ILV = 5
N_PAIRS = (N_CHUNKS // NUM_WORKERS + 1) // 2

_HI_MASK = -65536


def _sqrt16(x):
    i = lax.bitcast_convert_type(x, jnp.int32)
    y = lax.bitcast_convert_type(
        jnp.int32(0x5F3759DF) - lax.shift_right_arithmetic(i, 1), jnp.float32)
    half = jnp.float32(0.5) * x
    for _ in range(2):
        y = y * (jnp.float32(1.5) - half * y * y)
    return x * y


def _body(x_hbm, ei_hbm, ej_hbm, out_hbm,
          idx_l0, idx_r0, idx_l1, idx_r1,
          lhs0, rhs0, lhs1, rhs1, outb0, outb1,
          isem0, isem1, gsem0, gsem1, wsem0, wsem1):
    idx_l = (idx_l0, idx_l1)
    idx_r = (idx_r0, idx_r1)
    lhs = (lhs0, lhs1)
    rhs = (rhs0, rhs1)
    outb = (outb0, outb1)
    isem = (isem0, isem1)
    gsem = (gsem0, gsem1)
    wsem = (wsem0, wsem1)

    cid = lax.axis_index("c")
    sid = lax.axis_index("s")
    wid = sid * NUM_CORES + cid

    def svalid(s):
        return s * NUM_WORKERS + wid < N_CHUNKS

    def chunk_of(s):
        return s * NUM_WORKERS + wid

    def fire_idx(s, b):
        sl = pl.ds(chunk_of(s) * CHUNK, CHUNK)
        pltpu.async_copy(ei_hbm.at[sl], idx_l[b], isem[b])
        pltpu.async_copy(ej_hbm.at[sl], idx_r[b], isem[b])

    def fire_gathers(b):
        sl0 = pl.ds(0, CHUNK)
        pltpu.make_async_copy(ei_hbm.at[sl0], idx_l[b], isem[b]).wait()
        pltpu.make_async_copy(ej_hbm.at[sl0], idx_r[b], isem[b]).wait()
        pltpu.async_copy(x_hbm.at[idx_l[b]], lhs[b], gsem[b])
        pltpu.async_copy(x_hbm.at[idx_r[b]], rhs[b], gsem[b])

    def drain_gathers(b):
        pltpu.make_async_copy(x_hbm.at[idx_l[b]], lhs[b], gsem[b]).wait()
        pltpu.make_async_copy(x_hbm.at[idx_r[b]], rhs[b], gsem[b]).wait()

    def compute(s, b):
        def group_body(g2, _):
            for sub in range(GROUP_ILV):
                g = g2 * GROUP_ILV + sub
                rows = g * 16 + lax.iota(jnp.int32, 16)
                acc = jnp.zeros((16,), jnp.float32)
                for d2 in range(PACKED_DIM):
                    cols = jnp.full((16,), d2, jnp.int32)
                    a = plsc.load_gather(lhs[b], [rows, cols])
                    bb = plsc.load_gather(rhs[b], [rows, cols])
                    alo = lax.bitcast_convert_type(
                        lax.shift_left(a, 16), jnp.float32)
                    blo = lax.bitcast_convert_type(
                        lax.shift_left(bb, 16), jnp.float32)
                    ahi = lax.bitcast_convert_type(a & _HI_MASK, jnp.float32)
                    bhi = lax.bitcast_convert_type(bb & _HI_MASK, jnp.float32)
                    dlo = alo - blo
                    dhi = ahi - bhi
                    acc = acc + dlo * dlo
                    acc = acc + dhi * dhi
                outb[b][pl.ds(g * 16, 16)] = _sqrt16(acc + jnp.float32(1e-12))
            return ()

        lax.fori_loop(0, GROUPS // GROUP_ILV, group_body, (), unroll=False)
        pltpu.async_copy(outb[b], out_hbm.at[pl.ds(chunk_of(s) * CHUNK, CHUNK)],
                         wsem[b])

    def drain_write(b):
        pltpu.make_async_copy(outb[b], out_hbm.at[pl.ds(0, CHUNK)],
                              wsem[b]).wait()

    fire_idx(0, 0)

    @pl.when(svalid(1))
    def _():
        fire_idx(1, 1)

    fire_gathers(0)

    def pair_body(p, _):
        s = 2 * p

        @pl.when(svalid(s + 1))
        def _():
            fire_gathers(1)

        drain_gathers(0)

        @pl.when(svalid(s + 2))
        def _():
            fire_idx(s + 2, 0)

        @pl.when(p > 0)
        def _():
            drain_write(0)

        compute(s, 0)

        @pl.when(svalid(s + 2))
        def _():
            fire_gathers(0)

        @pl.when(svalid(s + 1))
        def _():
            drain_gathers(1)

            @pl.when(svalid(s + 3))
            def _():
                fire_idx(s + 3, 1)

            @pl.when(p > 0)
            def _():
                drain_write(1)

            compute(s + 1, 1)

        return ()

    lax.fori_loop(0, N_PAIRS, pair_body, (), unroll=False)

    drain_write(0)
    drain_write(1)


@jax.jit
def _mde_distances(x_packed, ei, ej):
    mesh = plsc.VectorSubcoreMesh(core_axis_name="c", subcore_axis_name="s")
    return pl.kernel(
        _body,
        out_type=jax.ShapeDtypeStruct((N_EDGES,), jnp.float32),
        mesh=mesh,
        compiler_params=pltpu.CompilerParams(
            needs_layout_passes=False, use_tc_tiling_on_sc=False),
        scratch_types=[
            pltpu.VMEM((CHUNK,), jnp.int32),
            pltpu.VMEM((CHUNK,), jnp.int32),
            pltpu.VMEM((CHUNK,), jnp.int32),
            pltpu.VMEM((CHUNK,), jnp.int32),
            pltpu.VMEM((CHUNK, PACKED_DIM), jnp.int32),
            pltpu.VMEM((CHUNK, PACKED_DIM), jnp.int32),
            pltpu.VMEM((CHUNK, PACKED_DIM), jnp.int32),
            pltpu.VMEM((CHUNK, PACKED_DIM), jnp.int32),
            pltpu.VMEM((CHUNK,), jnp.float32),
            pltpu.VMEM((CHUNK,), jnp.float32),
            pltpu.SemaphoreType.DMA,
            pltpu.SemaphoreType.DMA,
            pltpu.SemaphoreType.DMA,
            pltpu.SemaphoreType.DMA,
            pltpu.SemaphoreType.DMA,
            pltpu.SemaphoreType.DMA,
        ],
    )(x_packed, ei, ej)


def kernel(X, edges):
    edges = edges.astype(jnp.int32)
    ei = edges[:, 0]
    ej = edges[:, 1]
    xb = X.astype(jnp.bfloat16).reshape(N_ITEMS, PACKED_DIM, 2)
    x_packed = lax.bitcast_convert_type(xb, jnp.int32)
    return _mde_distances(x_packed, ei, ej)


# speedup vs baseline: 1.0870x; 1.0870x over previous
import jax
import jax.numpy as jnp
from jax import lax
from jax.experimental import pallas as pl
from jax.experimental.pallas import tpu as pltpu
from jax.experimental.pallas import tpu_sc as plsc

N_ITEMS = 100000
EMBED_DIM = 16
PACKED_DIM = EMBED_DIM // 2
N_EDGES = 3200000

NUM_CORES = 2
NUM_SUBCORES = 16
NUM_WORKERS = NUM_CORES * NUM_SUBCORES

CHUNK = 1024
N_CHUNKS = N_EDGES // CHUNK
GROUPS = CHUNK // 16
N_PAIRS = (N_CHUNKS // NUM_WORKERS + 1) // 2

_HI_MASK = -65536


def _sqrt16(x):
    i = lax.bitcast_convert_type(x, jnp.int32)
    y = lax.bitcast_convert_type(
        jnp.int32(0x5F3759DF) - lax.shift_right_arithmetic(i, 1), jnp.float32)
    half = jnp.float32(0.5) * x
    for _ in range(2):
        y = y * (jnp.float32(1.5) - half * y * y)
    return x * y


def _body(x_hbm, ei_hbm, ej_hbm, out_hbm,
          idx_l0, idx_r0, idx_l1, idx_r1,
          lhs0, rhs0, lhs1, rhs1, outb0, outb1,
          isem0, isem1, gsem0, gsem1, wsem0, wsem1):
    idx_l = (idx_l0, idx_l1)
    idx_r = (idx_r0, idx_r1)
    lhs = (lhs0, lhs1)
    rhs = (rhs0, rhs1)
    outb = (outb0, outb1)
    isem = (isem0, isem1)
    gsem = (gsem0, gsem1)
    wsem = (wsem0, wsem1)

    cid = lax.axis_index("c")
    sid = lax.axis_index("s")
    wid = sid * NUM_CORES + cid

    def svalid(s):
        return s * NUM_WORKERS + wid < N_CHUNKS

    def chunk_of(s):
        return s * NUM_WORKERS + wid

    def fire_idx(s, b):
        sl = pl.ds(chunk_of(s) * CHUNK, CHUNK)
        pltpu.async_copy(ei_hbm.at[sl], idx_l[b], isem[b])
        pltpu.async_copy(ej_hbm.at[sl], idx_r[b], isem[b])

    def fire_gathers(b):
        sl0 = pl.ds(0, CHUNK)
        pltpu.make_async_copy(ei_hbm.at[sl0], idx_l[b], isem[b]).wait()
        pltpu.make_async_copy(ej_hbm.at[sl0], idx_r[b], isem[b]).wait()
        pltpu.async_copy(x_hbm.at[idx_l[b]], lhs[b], gsem[b])
        pltpu.async_copy(x_hbm.at[idx_r[b]], rhs[b], gsem[b])

    def drain_gathers(b):
        pltpu.make_async_copy(x_hbm.at[idx_l[b]], lhs[b], gsem[b]).wait()
        pltpu.make_async_copy(x_hbm.at[idx_r[b]], rhs[b], gsem[b]).wait()

    def compute(s, b):
        def group_body(g2, _):
            for sub in range(2):
                g = g2 * 2 + sub
                rows = g * 16 + lax.iota(jnp.int32, 16)
                acc = jnp.zeros((16,), jnp.float32)
                for d2 in range(PACKED_DIM):
                    cols = jnp.full((16,), d2, jnp.int32)
                    a = plsc.load_gather(lhs[b], [rows, cols])
                    bb = plsc.load_gather(rhs[b], [rows, cols])
                    alo = lax.bitcast_convert_type(
                        lax.shift_left(a, 16), jnp.float32)
                    blo = lax.bitcast_convert_type(
                        lax.shift_left(bb, 16), jnp.float32)
                    ahi = lax.bitcast_convert_type(a & _HI_MASK, jnp.float32)
                    bhi = lax.bitcast_convert_type(bb & _HI_MASK, jnp.float32)
                    dlo = alo - blo
                    dhi = ahi - bhi
                    acc = acc + dlo * dlo
                    acc = acc + dhi * dhi
                outb[b][pl.ds(g * 16, 16)] = _sqrt16(acc + jnp.float32(1e-12))
            return ()

        lax.fori_loop(0, GROUPS // 2, group_body, (), unroll=False)
        pltpu.async_copy(outb[b], out_hbm.at[pl.ds(chunk_of(s) * CHUNK, CHUNK)],
                         wsem[b])

    def drain_write(b):
        pltpu.make_async_copy(outb[b], out_hbm.at[pl.ds(0, CHUNK)],
                              wsem[b]).wait()

    fire_idx(0, 0)

    @pl.when(svalid(1))
    def _():
        fire_idx(1, 1)

    fire_gathers(0)

    def pair_body(p, _):
        s = 2 * p

        @pl.when(svalid(s + 1))
        def _():
            fire_gathers(1)

        drain_gathers(0)

        @pl.when(svalid(s + 2))
        def _():
            fire_idx(s + 2, 0)

        @pl.when(p > 0)
        def _():
            drain_write(0)

        compute(s, 0)

        @pl.when(svalid(s + 2))
        def _():
            fire_gathers(0)

        @pl.when(svalid(s + 1))
        def _():
            drain_gathers(1)

            @pl.when(svalid(s + 3))
            def _():
                fire_idx(s + 3, 1)

            @pl.when(p > 0)
            def _():
                drain_write(1)

            compute(s + 1, 1)

        return ()

    lax.fori_loop(0, N_PAIRS, pair_body, (), unroll=False)

    drain_write(0)
    drain_write(1)


@jax.jit
def _mde_distances(x_packed, ei, ej):
    mesh = plsc.VectorSubcoreMesh(core_axis_name="c", subcore_axis_name="s")
    return pl.kernel(
        _body,
        out_type=jax.ShapeDtypeStruct((N_EDGES,), jnp.float32),
        mesh=mesh,
        compiler_params=pltpu.CompilerParams(
            needs_layout_passes=False, use_tc_tiling_on_sc=False),
        scratch_types=[
            pltpu.VMEM((CHUNK,), jnp.int32),
            pltpu.VMEM((CHUNK,), jnp.int32),
            pltpu.VMEM((CHUNK,), jnp.int32),
            pltpu.VMEM((CHUNK,), jnp.int32),
            pltpu.VMEM((CHUNK, PACKED_DIM), jnp.int32),
            pltpu.VMEM((CHUNK, PACKED_DIM), jnp.int32),
            pltpu.VMEM((CHUNK, PACKED_DIM), jnp.int32),
            pltpu.VMEM((CHUNK, PACKED_DIM), jnp.int32),
            pltpu.VMEM((CHUNK,), jnp.float32),
            pltpu.VMEM((CHUNK,), jnp.float32),
            pltpu.SemaphoreType.DMA,
            pltpu.SemaphoreType.DMA,
            pltpu.SemaphoreType.DMA,
            pltpu.SemaphoreType.DMA,
            pltpu.SemaphoreType.DMA,
            pltpu.SemaphoreType.DMA,
        ],
    )(x_packed, ei, ej)


def kernel(X, edges):
    edges = edges.astype(jnp.int32)
    ei = edges[:, 0]
    ej = edges[:, 1]
    xb = X.astype(jnp.bfloat16).reshape(N_ITEMS, PACKED_DIM, 2)
    x_packed = lax.bitcast_convert_type(xb, jnp.int32)
    return _mde_distances(x_packed, ei, ej)
